# baseline (speedup 1.0000x reference)
import functools

import jax
import jax.numpy as jnp
import numpy as np
from jax.experimental import pallas as pl
from jax.experimental.pallas import tpu as pltpu
from jax.experimental.shard_map import shard_map
from jax.sharding import Mesh, PartitionSpec

_DIM = 3
_THRESHOLD = 0.4
_J = 3
_N = 2048
_CHUNK = 256


def _block_kernel(xt_ref, out_ref, w_scr):
    xt = xt_ref[0]
    rn_row = jnp.sum(xt * xt, axis=0, keepdims=True)

    deg_row = jnp.zeros((1, _N), jnp.float32)
    for c in range(_N // _CHUNK):
        xtc = xt[:, c * _CHUNK:(c + 1) * _CHUNK]
        rn_col = rn_row[:, c * _CHUNK:(c + 1) * _CHUNK].T
        G = jax.lax.dot_general(
            xtc, xt, (((0,), (0,)), ((), ())),
            preferred_element_type=jnp.float32)
        D = rn_col + rn_row - 2.0 * G
        Wc = jnp.exp(-D)
        Wc = jnp.where(Wc >= _THRESHOLD, Wc, 0.0)
        deg_row = deg_row + jnp.sum(Wc, axis=0, keepdims=True)
        w_scr[c * _CHUNK:(c + 1) * _CHUNK, :] = Wc.astype(jnp.bfloat16)
    rdeg_row = 1.0 / jnp.maximum(deg_row, 1e-12)

    def apply(s):
        sp = (s * rdeg_row).astype(jnp.bfloat16)
        mv = jax.lax.dot_general(
            sp, w_scr[...], (((1,), (0,)), ((), ())),
            preferred_element_type=jnp.float32)
        return 0.5 * (s + mv)

    zeros13 = jnp.zeros((13, _N), jnp.float32)
    s_ = jnp.concatenate([xt, zeros13], axis=0)
    s_ = apply(s_)
    s1 = s_[0:3]
    u0 = jnp.abs(xt - s1)
    s_ = jnp.concatenate([s_[0:3], u0, s_[6:16]], axis=0)
    s_ = apply(s_)
    s2 = s_[0:3]
    u0s1 = s_[3:6]
    u1 = jnp.abs(s1 - s2)
    s_ = jnp.concatenate([s_[0:6], u1, s_[9:16]], axis=0)
    s_ = apply(s_)
    u0s2 = s_[3:6]
    s_ = apply(s_)
    s4, u1s2 = s_[0:3], s_[6:9]
    u2 = jnp.abs(s2 - s4)
    s_ = jnp.concatenate([s_[0:9], u2, s_[12:16]], axis=0)
    s_ = apply(s_)
    u0s4 = s_[3:6]
    s_ = apply(s_)
    u1s4 = s_[6:9]
    s_ = apply(s_)
    s_ = apply(s_)
    s8, u2s4 = s_[0:3], s_[9:12]
    s_ = apply(s_)
    u0s8 = s_[3:6]
    s_ = apply(s_)
    u1s8 = s_[6:9]
    s_ = apply(s_)
    s_ = apply(s_)
    u2s8 = s_[9:12]

    def pool(v):
        return jnp.sum(v, axis=1, keepdims=True)

    cols = [pool(s8)]
    cols.append(pool(u0))
    cols.append(pool(jnp.abs(u0s1 - u0s2)))
    cols.append(pool(jnp.abs(u0s2 - u0s4)))
    cols.append(pool(jnp.abs(u0s4 - u0s8)))
    cols.append(pool(u1))
    cols.append(pool(jnp.abs(u1s2 - u1s4)))
    cols.append(pool(jnp.abs(u1s4 - u1s8)))
    cols.append(pool(u2))
    cols.append(pool(jnp.abs(u2s4 - u2s8)))
    cols.append(pool(jnp.abs(s4 - s8)))
    out_ref[0] = jnp.concatenate(cols, axis=1)


_N_COLS = 1 + (_J + 1) + (_J + 1) * _J // 2


def _scatter_feats(xbt_local):
    nb, dim, n = xbt_local.shape
    return pl.pallas_call(
        _block_kernel,
        grid=(nb,),
        in_specs=[pl.BlockSpec((1, dim, n), lambda b: (b, 0, 0))],
        out_specs=pl.BlockSpec((1, dim, _N_COLS), lambda b: (b, 0, 0)),
        out_shape=jax.ShapeDtypeStruct((nb, dim, _N_COLS), jnp.float32),
        scratch_shapes=[pltpu.VMEM((_N, _N), jnp.bfloat16)],
    )(xbt_local)


@functools.partial(jax.jit, static_argnames=())
def kernel(point_clouds, sigma, alphas):
    b_pc, n, dim = point_clouds.shape
    nw = alphas.shape[0]
    sqrt_sigma = jnp.sqrt(sigma.astype(jnp.float32))
    scale = alphas / sqrt_sigma
    xbt = (point_clouds.transpose(0, 2, 1)[:, None, :, :]
           * scale[None, :, :, None]).reshape(b_pc * nw, dim, n)

    devs = jax.devices()
    n_dev = 2 if len(devs) >= 2 and (b_pc * nw) % 2 == 0 else 1
    if n_dev == 2:
        mesh = Mesh(np.array(devs[:2]), ("d",))
        out = shard_map(
            _scatter_feats, mesh=mesh,
            in_specs=(PartitionSpec("d", None, None),),
            out_specs=PartitionSpec("d", None, None),
            check_rep=False)(xbt)
    else:
        out = _scatter_feats(xbt)

    feats = out.transpose(0, 2, 1).reshape(b_pc * nw, _N_COLS * dim)
    return (feats * sqrt_sigma).reshape(b_pc, nw * _N_COLS * dim)


# speedup vs baseline: 2.9341x; 2.9341x over previous
import functools

import jax
import jax.numpy as jnp
from jax.experimental import pallas as pl
from jax.experimental.pallas import tpu as pltpu

_DIM = 3
_THRESHOLD = 0.4
_J = 3
_N = 2048
_CHUNK = 512


def _block_kernel(xt_ref, out_ref, w_scr):
    xt = xt_ref[0]
    rn_row = jnp.sum(xt * xt, axis=0, keepdims=True)

    deg_row = jnp.zeros((1, _N), jnp.float32)
    for c in range(_N // _CHUNK):
        xtc = xt[:, c * _CHUNK:(c + 1) * _CHUNK]
        rn_col = rn_row[:, c * _CHUNK:(c + 1) * _CHUNK].T
        G = jax.lax.dot_general(
            xtc, xt, (((0,), (0,)), ((), ())),
            preferred_element_type=jnp.float32)
        D = rn_col + rn_row - 2.0 * G
        Wc = jnp.exp(-D)
        Wc = jnp.where(Wc >= _THRESHOLD, Wc, 0.0)
        deg_row = deg_row + jnp.sum(Wc, axis=0, keepdims=True)
        w_scr[c * _CHUNK:(c + 1) * _CHUNK, :] = Wc.astype(jnp.bfloat16)
    rdeg_row = 1.0 / jnp.maximum(deg_row, 1e-12)

    def apply(s):
        sp = (s * rdeg_row).astype(jnp.bfloat16)
        mv = jax.lax.dot_general(
            sp, w_scr[...], (((1,), (0,)), ((), ())),
            preferred_element_type=jnp.float32)
        return 0.5 * (s + mv)

    zeros13 = jnp.zeros((13, _N), jnp.float32)
    s_ = jnp.concatenate([xt, zeros13], axis=0)
    s_ = apply(s_)
    s1 = s_[0:3]
    u0 = jnp.abs(xt - s1)
    s_ = jnp.concatenate([s_[0:3], u0, s_[6:16]], axis=0)
    s_ = apply(s_)
    s2 = s_[0:3]
    u0s1 = s_[3:6]
    u1 = jnp.abs(s1 - s2)
    s_ = jnp.concatenate([s_[0:6], u1, s_[9:16]], axis=0)
    s_ = apply(s_)
    u0s2 = s_[3:6]
    s_ = apply(s_)
    s4, u1s2 = s_[0:3], s_[6:9]
    u2 = jnp.abs(s2 - s4)
    s_ = jnp.concatenate([s_[0:9], u2, s_[12:16]], axis=0)
    s_ = apply(s_)
    u0s4 = s_[3:6]
    s_ = apply(s_)
    u1s4 = s_[6:9]
    s_ = apply(s_)
    s_ = apply(s_)
    s8, u2s4 = s_[0:3], s_[9:12]
    s_ = apply(s_)
    u0s8 = s_[3:6]
    s_ = apply(s_)
    u1s8 = s_[6:9]
    s_ = apply(s_)
    s_ = apply(s_)
    u2s8 = s_[9:12]

    def pool(v):
        return jnp.sum(v, axis=1, keepdims=True)

    cols = [pool(s8)]
    cols.append(pool(u0))
    cols.append(pool(jnp.abs(u0s1 - u0s2)))
    cols.append(pool(jnp.abs(u0s2 - u0s4)))
    cols.append(pool(jnp.abs(u0s4 - u0s8)))
    cols.append(pool(u1))
    cols.append(pool(jnp.abs(u1s2 - u1s4)))
    cols.append(pool(jnp.abs(u1s4 - u1s8)))
    cols.append(pool(u2))
    cols.append(pool(jnp.abs(u2s4 - u2s8)))
    cols.append(pool(jnp.abs(s4 - s8)))
    out_ref[0] = jnp.concatenate(cols, axis=1)


@functools.partial(jax.jit, static_argnames=())
def kernel(point_clouds, sigma, alphas):
    b_pc, n, dim = point_clouds.shape
    nw = alphas.shape[0]
    sqrt_sigma = jnp.sqrt(sigma.astype(jnp.float32))
    scale = alphas / sqrt_sigma
    xbt = (point_clouds.transpose(0, 2, 1)[:, None, :, :]
           * scale[None, :, :, None]).reshape(b_pc * nw, dim, n)

    n_cols = 1 + (_J + 1) + (_J + 1) * _J // 2
    out = pl.pallas_call(
        _block_kernel,
        grid=(b_pc * nw,),
        in_specs=[pl.BlockSpec((1, dim, n), lambda b: (b, 0, 0))],
        out_specs=pl.BlockSpec((1, dim, n_cols), lambda b: (b, 0, 0)),
        out_shape=jax.ShapeDtypeStruct((b_pc * nw, dim, n_cols), jnp.float32),
        scratch_shapes=[pltpu.VMEM((_N, _N), jnp.bfloat16)],
    )(xbt)

    feats = out.transpose(0, 2, 1).reshape(b_pc * nw, n_cols * dim)
    return (feats * sqrt_sigma).reshape(b_pc, nw * n_cols * dim)


# speedup vs baseline: 2.9368x; 1.0009x over previous
import functools

import jax
import jax.numpy as jnp
from jax.experimental import pallas as pl
from jax.experimental.pallas import tpu as pltpu

_DIM = 3
_THRESHOLD = 0.4
_J = 3
_N = 2048
_CHUNK = 1024


def _block_kernel(xt_ref, out_ref, w_scr):
    xt = xt_ref[0]
    rn_row = jnp.sum(xt * xt, axis=0, keepdims=True)

    deg_row = jnp.zeros((1, _N), jnp.float32)
    for c in range(_N // _CHUNK):
        xtc = xt[:, c * _CHUNK:(c + 1) * _CHUNK]
        rn_col = rn_row[:, c * _CHUNK:(c + 1) * _CHUNK].T
        G = jax.lax.dot_general(
            xtc, xt, (((0,), (0,)), ((), ())),
            preferred_element_type=jnp.float32)
        D = rn_col + rn_row - 2.0 * G
        Wc = jnp.exp(-D)
        Wc = jnp.where(Wc >= _THRESHOLD, Wc, 0.0)
        deg_row = deg_row + jnp.sum(Wc, axis=0, keepdims=True)
        w_scr[c * _CHUNK:(c + 1) * _CHUNK, :] = Wc.astype(jnp.bfloat16)
    rdeg_row = 1.0 / jnp.maximum(deg_row, 1e-12)

    def apply(s):
        sp = (s * rdeg_row).astype(jnp.bfloat16)
        mv = jax.lax.dot_general(
            sp, w_scr[...], (((1,), (0,)), ((), ())),
            preferred_element_type=jnp.float32)
        return 0.5 * (s + mv)

    zeros13 = jnp.zeros((13, _N), jnp.float32)
    s_ = jnp.concatenate([xt, zeros13], axis=0)
    s_ = apply(s_)
    s1 = s_[0:3]
    u0 = jnp.abs(xt - s1)
    s_ = jnp.concatenate([s_[0:3], u0, s_[6:16]], axis=0)
    s_ = apply(s_)
    s2 = s_[0:3]
    u0s1 = s_[3:6]
    u1 = jnp.abs(s1 - s2)
    s_ = jnp.concatenate([s_[0:6], u1, s_[9:16]], axis=0)
    s_ = apply(s_)
    u0s2 = s_[3:6]
    s_ = apply(s_)
    s4, u1s2 = s_[0:3], s_[6:9]
    u2 = jnp.abs(s2 - s4)
    s_ = jnp.concatenate([s_[0:9], u2, s_[12:16]], axis=0)
    s_ = apply(s_)
    u0s4 = s_[3:6]
    s_ = apply(s_)
    u1s4 = s_[6:9]
    s_ = apply(s_)
    s_ = apply(s_)
    s8, u2s4 = s_[0:3], s_[9:12]
    s_ = apply(s_)
    u0s8 = s_[3:6]
    s_ = apply(s_)
    u1s8 = s_[6:9]
    s_ = apply(s_)
    s_ = apply(s_)
    u2s8 = s_[9:12]

    def pool(v):
        return jnp.sum(v, axis=1, keepdims=True)

    cols = [pool(s8)]
    cols.append(pool(u0))
    cols.append(pool(jnp.abs(u0s1 - u0s2)))
    cols.append(pool(jnp.abs(u0s2 - u0s4)))
    cols.append(pool(jnp.abs(u0s4 - u0s8)))
    cols.append(pool(u1))
    cols.append(pool(jnp.abs(u1s2 - u1s4)))
    cols.append(pool(jnp.abs(u1s4 - u1s8)))
    cols.append(pool(u2))
    cols.append(pool(jnp.abs(u2s4 - u2s8)))
    cols.append(pool(jnp.abs(s4 - s8)))
    out_ref[0] = jnp.concatenate(cols, axis=1)


@functools.partial(jax.jit, static_argnames=())
def kernel(point_clouds, sigma, alphas):
    b_pc, n, dim = point_clouds.shape
    nw = alphas.shape[0]
    sqrt_sigma = jnp.sqrt(sigma.astype(jnp.float32))
    scale = alphas / sqrt_sigma
    xbt = (point_clouds.transpose(0, 2, 1)[:, None, :, :]
           * scale[None, :, :, None]).reshape(b_pc * nw, dim, n)

    n_cols = 1 + (_J + 1) + (_J + 1) * _J // 2
    out = pl.pallas_call(
        _block_kernel,
        grid=(b_pc * nw,),
        in_specs=[pl.BlockSpec((1, dim, n), lambda b: (b, 0, 0))],
        out_specs=pl.BlockSpec((1, dim, n_cols), lambda b: (b, 0, 0)),
        out_shape=jax.ShapeDtypeStruct((b_pc * nw, dim, n_cols), jnp.float32),
        scratch_shapes=[pltpu.VMEM((_N, _N), jnp.bfloat16)],
    )(xbt)

    feats = out.transpose(0, 2, 1).reshape(b_pc * nw, n_cols * dim)
    return (feats * sqrt_sigma).reshape(b_pc, nw * n_cols * dim)
